# baseline (speedup 1.0000x reference)
import functools

import jax
import jax.numpy as jnp
from jax import lax
from jax.experimental import pallas as pl
from jax.experimental.pallas import tpu as pltpu
from jax.experimental.pallas import tpu_sc as plsc

BATCH = 4
SEQ_LEN = 2048
EMBED_DIM = 768

NC = 2
NS = 16
NW = NC * NS

ROWS = BATCH * SEQ_LEN
FLAT = ROWS * EMBED_DIM
PER_W = FLAT // NW
POS_FLAT = SEQ_LEN * EMBED_DIM
CHUNK = 16384
NCHUNK = PER_W // CHUNK
LANES = 16

_mesh = plsc.VectorSubcoreMesh(core_axis_name="c", subcore_axis_name="s")


@functools.partial(
    pl.kernel,
    out_type=jax.ShapeDtypeStruct((FLAT,), jnp.float32),
    mesh=_mesh,
    scratch_types=[
        pltpu.VMEM((2, CHUNK), jnp.float32),
        pltpu.VMEM((2, CHUNK), jnp.float32),
        pltpu.VMEM((2, CHUNK), jnp.float32),
        pltpu.SemaphoreType.DMA,
        pltpu.SemaphoreType.DMA,
        pltpu.SemaphoreType.DMA,
        pltpu.SemaphoreType.DMA,
        pltpu.SemaphoreType.DMA,
        pltpu.SemaphoreType.DMA,
    ],
)
def _sc_pos_add(x_hbm, pos_hbm, out_hbm, xb, pb, ob, sx0, sx1, sp0, sp1, so0, so1):
    wid = lax.axis_index("s") * NC + lax.axis_index("c")
    base = wid * PER_W
    p_base = lax.rem(base, POS_FLAT)
    sx = (sx0, sx1)
    sp = (sp0, sp1)
    so = (so0, so1)

    def load(c):
        k = c % 2
        off = c * CHUNK
        pltpu.async_copy(x_hbm.at[pl.ds(base + off, CHUNK)], xb.at[k], sx[k])
        pltpu.async_copy(pos_hbm.at[pl.ds(p_base + off, CHUNK)], pb.at[k], sp[k])

    load(0)
    for c in range(NCHUNK):
        k = c % 2
        off = c * CHUNK
        pltpu.make_async_copy(x_hbm.at[pl.ds(base + off, CHUNK)], xb.at[k], sx[k]).wait()
        pltpu.make_async_copy(pos_hbm.at[pl.ds(p_base + off, CHUNK)], pb.at[k], sp[k]).wait()
        if c + 1 < NCHUNK:
            load(c + 1)
        if c >= 2:
            po = (c - 2) * CHUNK
            pltpu.make_async_copy(
                ob.at[k], out_hbm.at[pl.ds(base + po, CHUNK)], so[k]
            ).wait()

        @plsc.parallel_loop(0, CHUNK, step=LANES, unroll=8)
        def _add(i):
            ob[k, pl.ds(i, LANES)] = xb[k, pl.ds(i, LANES)] + pb[k, pl.ds(i, LANES)]

        pltpu.async_copy(ob.at[k], out_hbm.at[pl.ds(base + off, CHUNK)], so[k])

    for c in (NCHUNK - 2, NCHUNK - 1):
        k = c % 2
        off = c * CHUNK
        pltpu.make_async_copy(ob.at[k], out_hbm.at[pl.ds(base + off, CHUNK)], so[k]).wait()


def kernel(x, pos_table):
    out = _sc_pos_add(x.reshape(FLAT), pos_table.reshape(POS_FLAT))
    return out.reshape(BATCH, SEQ_LEN, EMBED_DIM)


# speedup vs baseline: 1.3368x; 1.3368x over previous
import functools

import jax
import jax.numpy as jnp
from jax import lax
from jax.experimental import pallas as pl
from jax.experimental.pallas import tpu as pltpu
from jax.experimental.pallas import tpu_sc as plsc

BATCH = 4
SEQ_LEN = 2048
EMBED_DIM = 768
SPLIT = 1792

NC = 2
NS = 16
NW = NC * NS

SC_SEQ = SEQ_LEN - SPLIT
SC_ROWS = BATCH * SC_SEQ
WPB = NW // BATCH
RPW = SC_SEQ // WPB
CHUNK = RPW * EMBED_DIM
HALF = CHUNK // 2
LANES = 16

_mesh = plsc.VectorSubcoreMesh(core_axis_name="c", subcore_axis_name="s")


@functools.partial(
    pl.kernel,
    out_type=jax.ShapeDtypeStruct((SC_ROWS * EMBED_DIM,), jnp.float32),
    mesh=_mesh,
    scratch_types=[
        pltpu.VMEM((2, HALF), jnp.float32),
        pltpu.VMEM((2, HALF), jnp.float32),
        pltpu.VMEM((2, HALF), jnp.float32),
        pltpu.SemaphoreType.DMA,
        pltpu.SemaphoreType.DMA,
        pltpu.SemaphoreType.DMA,
        pltpu.SemaphoreType.DMA,
        pltpu.SemaphoreType.DMA,
        pltpu.SemaphoreType.DMA,
    ],
)
def _sc_tail_add(x_hbm, pos_hbm, out_hbm, xb, pb, ob, sx0, sx1, sp0, sp1, so0, so1):
    wid = lax.axis_index("s") * NC + lax.axis_index("c")
    b = wid // WPB
    j = lax.rem(wid, WPB)
    x_base = (b * SEQ_LEN + SPLIT + j * RPW) * EMBED_DIM
    p_base = (SPLIT + j * RPW) * EMBED_DIM
    o_base = wid * CHUNK
    sx = (sx0, sx1)
    sp = (sp0, sp1)
    so = (so0, so1)

    def load(c):
        pltpu.async_copy(x_hbm.at[pl.ds(x_base + c * HALF, HALF)], xb.at[c], sx[c])
        pltpu.async_copy(pos_hbm.at[pl.ds(p_base + c * HALF, HALF)], pb.at[c], sp[c])

    load(0)
    load(1)
    for c in range(2):
        pltpu.make_async_copy(
            x_hbm.at[pl.ds(x_base + c * HALF, HALF)], xb.at[c], sx[c]
        ).wait()
        pltpu.make_async_copy(
            pos_hbm.at[pl.ds(p_base + c * HALF, HALF)], pb.at[c], sp[c]
        ).wait()

        @plsc.parallel_loop(0, HALF, step=LANES, unroll=8)
        def _add(i):
            ob[c, pl.ds(i, LANES)] = xb[c, pl.ds(i, LANES)] + pb[c, pl.ds(i, LANES)]

        pltpu.async_copy(ob.at[c], out_hbm.at[pl.ds(o_base + c * HALF, HALF)], so[c])

    for c in range(2):
        pltpu.make_async_copy(
            ob.at[c], out_hbm.at[pl.ds(o_base + c * HALF, HALF)], so[c]
        ).wait()


def _tc_add_kernel(x_ref, pos_ref, o_ref):
    o_ref[...] = x_ref[...] + pos_ref[...]


def _tc_head(x, pos_table):
    return pl.pallas_call(
        _tc_add_kernel,
        grid=(BATCH // 2,),
        in_specs=[
            pl.BlockSpec((2, SPLIT, EMBED_DIM), lambda b: (b, 0, 0)),
            pl.BlockSpec((SPLIT, EMBED_DIM), lambda b: (0, 0)),
        ],
        out_specs=pl.BlockSpec((2, SPLIT, EMBED_DIM), lambda b: (b, 0, 0)),
        out_shape=jax.ShapeDtypeStruct((BATCH, SPLIT, EMBED_DIM), x.dtype),
    )(x, pos_table)


def kernel(x, pos_table):
    head = _tc_head(x, pos_table)
    tail = _sc_tail_add(x.reshape(-1), pos_table.reshape(-1))
    tail = tail.reshape(BATCH, SC_SEQ, EMBED_DIM)
    return jnp.concatenate([head, tail], axis=1)


# speedup vs baseline: 6.6780x; 4.9954x over previous
import jax
import jax.numpy as jnp
from jax.experimental import pallas as pl

BATCH = 4
SEQ_LEN = 2048
EMBED_DIM = 768


def _add_kernel(x_ref, pos_ref, o_ref):
    o_ref[...] = x_ref[...] + pos_ref[...]


def kernel(x, pos_table):
    grid = (BATCH // 2,)
    return pl.pallas_call(
        _add_kernel,
        grid=grid,
        in_specs=[
            pl.BlockSpec((2, SEQ_LEN, EMBED_DIM), lambda b: (b, 0, 0)),
            pl.BlockSpec((SEQ_LEN, EMBED_DIM), lambda b: (0, 0)),
        ],
        out_specs=pl.BlockSpec((2, SEQ_LEN, EMBED_DIM), lambda b: (b, 0, 0)),
        out_shape=jax.ShapeDtypeStruct(x.shape, x.dtype),
    )(x, pos_table)
